# Initial kernel scaffold; baseline (speedup 1.0000x reference)
import jax
import jax.numpy as jnp
from jax.experimental import pallas as pl


def kernel(x, table):
    raise NotImplementedError("write your pallas kernel here")


# speedup vs baseline: 1.8899x; 1.8899x over previous
import functools

import jax
import jax.numpy as jnp
from jax import lax
from jax.experimental import pallas as pl
from jax.experimental.pallas import tpu as pltpu
from jax.experimental.pallas import tpu_sc as plsc

D = 64
B_TOTAL = 16384 * 50
NC, NS = 2, 16
NW = NC * NS
CH = 128
ROWS_PER_W = B_TOTAL // NW
CHUNKS_PER_W = ROWS_PER_W // CH
NBUF = 4
GROUPS = CHUNKS_PER_W // NBUF

_mesh = plsc.VectorSubcoreMesh(core_axis_name="c", subcore_axis_name="s")


@functools.partial(
    pl.kernel,
    mesh=_mesh,
    out_type=jax.ShapeDtypeStruct((B_TOTAL, D), jnp.float32),
    compiler_params=pltpu.CompilerParams(use_tc_tiling_on_sc=False),
    scratch_types=[
        pltpu.VMEM((CHUNKS_PER_W, CH), jnp.int32),
        pltpu.VMEM((NBUF, CH, D), jnp.float32),
        pltpu.SemaphoreType.DMA,
        pltpu.SemaphoreType.DMA,
        pltpu.SemaphoreType.DMA,
        pltpu.SemaphoreType.DMA,
    ],
)
def _gather_kernel(table_hbm, idx_hbm, out_hbm, idx_v, rows_v, s0, s1, s2, s3):
    sems = (s0, s1, s2, s3)
    wid = lax.axis_index("s") * NC + lax.axis_index("c")
    chunk0 = wid * CHUNKS_PER_W
    out0 = wid * ROWS_PER_W

    pltpu.sync_copy(idx_hbm.at[pl.ds(chunk0, CHUNKS_PER_W)], idx_v)

    def start(b, c):
        pltpu.make_async_copy(
            table_hbm.at[idx_v.at[c]], rows_v.at[b], sems[b]).start()

    def wait(b, c):
        pltpu.make_async_copy(
            table_hbm.at[idx_v.at[c]], rows_v.at[b], sems[b]).wait()

    for b in range(NBUF):
        start(b, b)

    def body(g, carry):
        for b in range(NBUF):
            c = g * NBUF + b
            wait(b, c)
            pltpu.sync_copy(rows_v.at[b],
                            out_hbm.at[pl.ds(out0 + c * CH, CH)])

            @pl.when(g < GROUPS - 1)
            def _():
                start(b, c + NBUF)
        return carry

    lax.fori_loop(0, GROUPS, body, 0)


def kernel(x, table):
    xf = x.reshape(NW * CHUNKS_PER_W, CH).astype(jnp.int32)
    out = _gather_kernel(table, xf)
    return out.reshape(x.shape[0], x.shape[1], D)
